# Initial kernel scaffold; baseline (speedup 1.0000x reference)
import jax
import jax.numpy as jnp
from jax.experimental import pallas as pl


def kernel(s, targets, W, b, bins):
    raise NotImplementedError("write your pallas kernel here")


# speedup vs baseline: 13.9828x; 13.9828x over previous
import functools

import jax
import jax.numpy as jnp
from jax.experimental import pallas as pl

_B = 16384
_DM = 64
_DD = 26
_NB = 100
_LAMB = 0.1
_BB = 256


def _fused_kernel(s_ref, t_ref, wt_ref, bt_ref, bins_ref, logp_ref, loss_ref):
    s = s_ref[...]
    bins = bins_ref[...]
    widths = bins[:, 1:] - bins[:, :-1]
    bw = bt_ref[...] - jnp.log(widths)
    lo = bins[:, 0:1]
    hi = bins[:, _NB:_NB + 1]
    ci = jax.lax.broadcasted_iota(jnp.int32, (_BB, _NB), 1)

    num = jnp.zeros((_BB, 1), jnp.float32)
    den = jnp.zeros((_BB, 1), jnp.float32)
    for d in range(_DD):
        t = t_ref[:, d:d + 1]
        logits = jax.lax.dot_general(
            s, wt_ref[d], (((1,), (0,)), ((), ())),
            preferred_element_type=jnp.float32)
        u = logits + bw[d:d + 1, :]
        m = jnp.max(u, axis=1, keepdims=True)
        e = jnp.exp(u - m)
        z = jnp.sum(e, axis=1, keepdims=True)
        logp = u - (m + jnp.log(z))
        logp_ref[:, d, :] = logp

        diffs = logp[:, 1:] - logp[:, :-1]
        spik = (jnp.sum(diffs * diffs, axis=1, keepdims=True) / (_NB - 1)) / (
            jnp.sum(jnp.abs(logp), axis=1, keepdims=True) / _NB)

        cnt = jnp.sum((bins < t).astype(jnp.int32), axis=1, keepdims=True)
        idx = cnt - 1
        idx = jnp.where(t <= lo, 0, idx)
        idx = jnp.where(t >= hi, _NB - 1, idx)
        idx = jnp.clip(idx, 0, _NB - 1)
        logp_t = jnp.sum(jnp.where(ci == idx, logp, 0.0), axis=1,
                         keepdims=True)

        msk = (t != 0.0).astype(jnp.float32)
        num += (-logp_t + _LAMB * spik) * msk
        den += msk

    per_ex = num / jnp.maximum(den, 1.0)
    part = jnp.sum(per_ex, axis=0, keepdims=True) * (1.0 / _B)
    loss_ref[0, :, :] = jnp.broadcast_to(part, (1, 128))


@jax.jit
def _run(s, targets, wt, bt, bins2):
    nblk = _B // _BB
    logp, partials = pl.pallas_call(
        _fused_kernel,
        grid=(nblk,),
        in_specs=[
            pl.BlockSpec((_BB, _DM), lambda i: (i, 0)),
            pl.BlockSpec((_BB, _DD), lambda i: (i, 0)),
            pl.BlockSpec((_DD, _DM, _NB), lambda i: (0, 0, 0)),
            pl.BlockSpec((_DD, _NB), lambda i: (0, 0)),
            pl.BlockSpec((1, _NB + 1), lambda i: (0, 0)),
        ],
        out_specs=[
            pl.BlockSpec((_BB, _DD, _NB), lambda i: (i, 0, 0)),
            pl.BlockSpec((1, 1, 128), lambda i: (i, 0, 0)),
        ],
        out_shape=[
            jax.ShapeDtypeStruct((_B, _DD, _NB), jnp.float32),
            jax.ShapeDtypeStruct((nblk, 1, 128), jnp.float32),
        ],
    )(s, targets, wt, bt, bins2)
    loss = jnp.sum(partials[:, 0, 0])
    return loss, logp


def kernel(s, targets, W, b, bins):
    wt = jnp.transpose(W, (1, 2, 0))
    bt = jnp.transpose(b)
    bins2 = jnp.reshape(bins, (1, _NB + 1))
    return _run(s, targets, wt, bt, bins2)


# speedup vs baseline: 17.8172x; 1.2742x over previous
import functools

import jax
import jax.numpy as jnp
from jax.experimental import pallas as pl

_B = 16384
_DM = 64
_DD = 26
_NB = 100
_LAMB = 0.1
_BB = 256


def _fused_kernel(s_ref, t_ref, wt_ref, bt_ref, bins_ref, logp_ref, loss_ref):
    s = s_ref[...]
    bins = bins_ref[...]
    widths = bins[:, 1:] - bins[:, :-1]
    bw = bt_ref[...] - jnp.log(widths)
    ci = jax.lax.broadcasted_iota(jnp.int32, (1, _NB), 1)
    blo = jnp.where(ci == 0, jnp.float32(-3.4e38), bins[:, :_NB])
    bhi = jnp.where(ci == _NB - 1, jnp.float32(3.4e38), bins[:, 1:])

    num = jnp.zeros((_BB, 1), jnp.float32)
    den = jnp.zeros((_BB, 1), jnp.float32)
    for d in range(_DD):
        t = t_ref[:, d:d + 1]
        logits = jax.lax.dot_general(
            s, wt_ref[d], (((1,), (0,)), ((), ())),
            preferred_element_type=jnp.float32)
        u = logits + bw[d:d + 1, :]
        du = u[:, 1:] - u[:, :-1]
        sdq = jnp.sum(du * du, axis=1, keepdims=True)
        sel = (blo < t) & (t <= bhi)
        selu = jnp.sum(jnp.where(sel, u, 0.0), axis=1, keepdims=True)

        m = jnp.max(u, axis=1, keepdims=True)
        e = jnp.exp(u - m)
        z = jnp.sum(e, axis=1, keepdims=True)
        lz = m + jnp.log(z)
        logp = u - lz
        logp_ref[:, d, :] = logp
        salp = jnp.sum(jnp.abs(logp), axis=1, keepdims=True)

        spik = (sdq * (1.0 / (_NB - 1))) / (salp * (1.0 / _NB))
        logp_t = selu - lz
        msk = (t != 0.0).astype(jnp.float32)
        num += (-logp_t + _LAMB * spik) * msk
        den += msk

    per_ex = num / jnp.maximum(den, 1.0)
    part = jnp.sum(per_ex, axis=0, keepdims=True) * (1.0 / _B)
    loss_ref[0, :, :] = jnp.broadcast_to(part, (1, 128))


@jax.jit
def _run(s, targets, wt, bt, bins2):
    nblk = _B // _BB
    logp, partials = pl.pallas_call(
        _fused_kernel,
        grid=(nblk,),
        in_specs=[
            pl.BlockSpec((_BB, _DM), lambda i: (i, 0)),
            pl.BlockSpec((_BB, _DD), lambda i: (i, 0)),
            pl.BlockSpec((_DD, _DM, _NB), lambda i: (0, 0, 0)),
            pl.BlockSpec((_DD, _NB), lambda i: (0, 0)),
            pl.BlockSpec((1, _NB + 1), lambda i: (0, 0)),
        ],
        out_specs=[
            pl.BlockSpec((_BB, _DD, _NB), lambda i: (i, 0, 0)),
            pl.BlockSpec((1, 1, 128), lambda i: (i, 0, 0)),
        ],
        out_shape=[
            jax.ShapeDtypeStruct((_B, _DD, _NB), jnp.float32),
            jax.ShapeDtypeStruct((nblk, 1, 128), jnp.float32),
        ],
    )(s, targets, wt, bt, bins2)
    loss = jnp.sum(partials[:, 0, 0])
    return loss, logp


def kernel(s, targets, W, b, bins):
    wt = jnp.transpose(W, (1, 2, 0))
    bt = jnp.transpose(b)
    bins2 = jnp.reshape(bins, (1, _NB + 1))
    return _run(s, targets, wt, bt, bins2)


# speedup vs baseline: 34.0643x; 1.9119x over previous
import functools

import jax
import jax.numpy as jnp
from jax.experimental import pallas as pl
from jax.experimental.pallas import tpu as pltpu

_B = 16384
_DM = 64
_DD = 26
_NB = 100
_LAMB = 0.1
_BB = 512


def _fused_kernel(s_ref, t_ref, wt_ref, bt_ref, bins_ref, logp_ref, loss_ref,
                  acc):
    s = s_ref[...]
    bins = bins_ref[...]
    widths = bins[:, 1:] - bins[:, :-1]
    bw = bt_ref[...] - jnp.log(widths)
    ci = jax.lax.broadcasted_iota(jnp.int32, (1, _NB), 1)
    blo = jnp.where(ci == 0, jnp.float32(-3.4e38), bins[:, :_NB])
    bhi = jnp.where(ci == _NB - 1, jnp.float32(3.4e38), bins[:, 1:])
    ones_nb = jnp.ones((_NB, 128), jnp.float32)
    ones_nb1 = jnp.ones((_NB - 1, 128), jnp.float32)
    dims = (((1,), (0,)), ((), ()))

    for d in range(_DD):
        t = t_ref[:, d:d + 1]
        logits = jax.lax.dot_general(
            s, wt_ref[d], (((1,), (0,)), ((), ())),
            preferred_element_type=jnp.float32)
        u = logits + bw[d:d + 1, :]
        du = u[:, 1:] - u[:, :-1]
        sdq = jax.lax.dot_general(du * du, ones_nb1, dims,
                                  preferred_element_type=jnp.float32)
        sel = (blo < t) & (t <= bhi)
        selu = jax.lax.dot_general(jnp.where(sel, u, 0.0), ones_nb, dims,
                                   preferred_element_type=jnp.float32)

        m = jnp.max(u, axis=1, keepdims=True)
        e = jnp.exp(u - m)
        z = jnp.sum(e, axis=1, keepdims=True)
        lz = m + jnp.log(z)
        logp = u - lz
        logp_ref[:, d * _NB:(d + 1) * _NB] = logp
        salp = jax.lax.dot_general(jnp.abs(logp), ones_nb, dims,
                                   preferred_element_type=jnp.float32)

        acc[0, :, d:d + 1] = selu[:, d:d + 1] - lz
        acc[1, :, d:d + 1] = sdq[:, d:d + 1]
        acc[2, :, d:d + 1] = salp[:, d:d + 1]

    glp = acc[0, :, :_DD]
    sdqp = acc[1, :, :_DD]
    salpp = acc[2, :, :_DD]
    spikp = (sdqp * (_NB / (_NB - 1.0))) / salpp
    mskp = (t_ref[...] != 0.0).astype(jnp.float32)
    lossp = (_LAMB * spikp - glp) * mskp
    nump = jnp.sum(lossp, axis=1, keepdims=True)
    denp = jnp.sum(mskp, axis=1, keepdims=True)
    per_ex = nump / jnp.maximum(denp, 1.0)
    part = jnp.sum(per_ex, axis=0, keepdims=True) * (1.0 / _B)
    loss_ref[0, :, :] = jnp.broadcast_to(part, (1, 128))


@jax.jit
def _run(s, targets, wt, bt, bins2):
    nblk = _B // _BB
    logp, partials = pl.pallas_call(
        _fused_kernel,
        grid=(nblk,),
        in_specs=[
            pl.BlockSpec((_BB, _DM), lambda i: (i, 0)),
            pl.BlockSpec((_BB, _DD), lambda i: (i, 0)),
            pl.BlockSpec((_DD, _DM, _NB), lambda i: (0, 0, 0)),
            pl.BlockSpec((_DD, _NB), lambda i: (0, 0)),
            pl.BlockSpec((1, _NB + 1), lambda i: (0, 0)),
        ],
        out_specs=[
            pl.BlockSpec((_BB, _DD * _NB), lambda i: (i, 0)),
            pl.BlockSpec((1, 1, 128), lambda i: (i, 0, 0)),
        ],
        out_shape=[
            jax.ShapeDtypeStruct((_B, _DD * _NB), jnp.float32),
            jax.ShapeDtypeStruct((nblk, 1, 128), jnp.float32),
        ],
        scratch_shapes=[
            pltpu.VMEM((3, _BB, 128), jnp.float32),
        ],
        compiler_params=pltpu.CompilerParams(
            dimension_semantics=("arbitrary",)),
    )(s, targets, wt, bt, bins2)
    loss = jnp.sum(partials[:, 0, 0])
    return loss, jnp.reshape(logp, (_B, _DD, _NB))


def kernel(s, targets, W, b, bins):
    wt = jnp.transpose(W, (1, 2, 0))
    bt = jnp.transpose(b)
    bins2 = jnp.reshape(bins, (1, _NB + 1))
    return _run(s, targets, wt, bt, bins2)


# speedup vs baseline: 44.4982x; 1.3063x over previous
import functools

import jax
import jax.numpy as jnp
from jax.experimental import pallas as pl
from jax.experimental.pallas import tpu as pltpu

_B = 16384
_DM = 64
_DD = 26
_NB = 100
_LAMB = 0.1
_BB = 1024


def _fused_kernel(s_ref, t_ref, wt_ref, bt_ref, bins_ref, logp_hbm, loss_ref,
                  scr, acc, sem):
    i = pl.program_id(0)
    nblk = pl.num_programs(0)
    s = s_ref[...]
    bins = bins_ref[...]
    widths = bins[:, 1:] - bins[:, :-1]
    bw = bt_ref[...] - jnp.log(widths)
    ci = jax.lax.broadcasted_iota(jnp.int32, (1, _NB), 1)
    blo = jnp.where(ci == 0, jnp.float32(-3.4e38), bins[:, :_NB])
    bhi = jnp.where(ci == _NB - 1, jnp.float32(3.4e38), bins[:, 1:])
    ones_nb = jnp.ones((_NB, 128), jnp.float32)
    ones_nb1 = jnp.ones((_NB - 1, 128), jnp.float32)
    dims = (((1,), (0,)), ((), ()))

    @pl.when(i > 0)
    def _wait_prev():
        for d in range(_DD):
            pltpu.make_async_copy(
                scr.at[d], logp_hbm.at[pl.ds((i - 1) * _BB, _BB), d, :],
                sem).wait()

    for d in range(_DD):
        t = t_ref[:, d:d + 1]
        logits = jax.lax.dot_general(
            s, wt_ref[d], (((1,), (0,)), ((), ())),
            preferred_element_type=jnp.float32)
        u = logits + bw[d:d + 1, :]
        du = u[:, 1:] - u[:, :-1]
        sdq = jax.lax.dot_general(du * du, ones_nb1, dims,
                                  preferred_element_type=jnp.float32)
        sel = (blo < t) & (t <= bhi)
        selu = jax.lax.dot_general(jnp.where(sel, u, 0.0), ones_nb, dims,
                                   preferred_element_type=jnp.float32)

        m = jnp.max(u, axis=1, keepdims=True)
        e = jnp.exp(u - m)
        z = jnp.sum(e, axis=1, keepdims=True)
        lz = m + jnp.log(z)
        logp = u - lz
        scr[d, :, :] = logp
        pltpu.make_async_copy(
            scr.at[d], logp_hbm.at[pl.ds(i * _BB, _BB), d, :], sem).start()
        salp = jax.lax.dot_general(jnp.abs(logp), ones_nb, dims,
                                   preferred_element_type=jnp.float32)

        acc[0, :, d:d + 1] = selu[:, d:d + 1] - lz
        acc[1, :, d:d + 1] = sdq[:, d:d + 1]
        acc[2, :, d:d + 1] = salp[:, d:d + 1]

    glp = acc[0, :, :_DD]
    sdqp = acc[1, :, :_DD]
    salpp = acc[2, :, :_DD]
    spikp = (sdqp * (_NB / (_NB - 1.0))) / salpp
    mskp = (t_ref[...] != 0.0).astype(jnp.float32)
    lossp = (_LAMB * spikp - glp) * mskp
    nump = jnp.sum(lossp, axis=1, keepdims=True)
    denp = jnp.sum(mskp, axis=1, keepdims=True)
    per_ex = nump / jnp.maximum(denp, 1.0)
    part = jnp.sum(per_ex, axis=0, keepdims=True) * (1.0 / _B)
    loss_ref[0, :, :] = jnp.broadcast_to(part, (1, 128))

    @pl.when(i == nblk - 1)
    def _drain():
        for d in range(_DD):
            pltpu.make_async_copy(
                scr.at[d], logp_hbm.at[pl.ds(i * _BB, _BB), d, :],
                sem).wait()


@jax.jit
def _run(s, targets, wt, bt, bins2):
    nblk = _B // _BB
    logp, partials = pl.pallas_call(
        _fused_kernel,
        grid=(nblk,),
        in_specs=[
            pl.BlockSpec((_BB, _DM), lambda i: (i, 0)),
            pl.BlockSpec((_BB, _DD), lambda i: (i, 0)),
            pl.BlockSpec((_DD, _DM, _NB), lambda i: (0, 0, 0)),
            pl.BlockSpec((_DD, _NB), lambda i: (0, 0)),
            pl.BlockSpec((1, _NB + 1), lambda i: (0, 0)),
        ],
        out_specs=[
            pl.BlockSpec(memory_space=pltpu.MemorySpace.HBM),
            pl.BlockSpec((1, 1, 128), lambda i: (i, 0, 0)),
        ],
        out_shape=[
            jax.ShapeDtypeStruct((_B, _DD, _NB), jnp.float32),
            jax.ShapeDtypeStruct((nblk, 1, 128), jnp.float32),
        ],
        scratch_shapes=[
            pltpu.VMEM((_DD, _BB, _NB), jnp.float32),
            pltpu.VMEM((3, _BB, 128), jnp.float32),
            pltpu.SemaphoreType.DMA,
        ],
        compiler_params=pltpu.CompilerParams(
            dimension_semantics=("arbitrary",)),
    )(s, targets, wt, bt, bins2)
    loss = jnp.sum(partials[:, 0, 0])
    return loss, logp


def kernel(s, targets, W, b, bins):
    wt = jnp.transpose(W, (1, 2, 0))
    bt = jnp.transpose(b)
    bins2 = jnp.reshape(bins, (1, _NB + 1))
    return _run(s, targets, wt, bt, bins2)


# speedup vs baseline: 45.9358x; 1.0323x over previous
import functools

import jax
import jax.numpy as jnp
from jax.experimental import pallas as pl
from jax.experimental.pallas import tpu as pltpu

_B = 16384
_DM = 64
_DD = 26
_NB = 100
_LAMB = 0.1
_BB = 1024


def _fused_kernel(s_ref, t_ref, wt_ref, bt_ref, bins_ref, logp_hbm, loss_ref,
                  scr, acc, sem):
    i = pl.program_id(0)
    nblk = pl.num_programs(0)
    s = s_ref[...]
    bins = bins_ref[...]
    widths = bins[:, 1:] - bins[:, :-1]
    bw = bt_ref[...] - jnp.log(widths)
    ci = jax.lax.broadcasted_iota(jnp.int32, (1, _NB), 1)
    blo = jnp.where(ci == 0, jnp.float32(-3.4e38), bins[:, :_NB])
    bhi = jnp.where(ci == _NB - 1, jnp.float32(3.4e38), bins[:, 1:])
    ones_nb = jnp.ones((_NB, 128), jnp.bfloat16)
    ones_nb1 = jnp.ones((_NB - 1, 128), jnp.bfloat16)
    dims = (((1,), (0,)), ((), ()))

    @pl.when(i > 0)
    def _wait_prev():
        for d in range(_DD):
            pltpu.make_async_copy(
                scr.at[d], logp_hbm.at[pl.ds((i - 1) * _BB, _BB), d, :],
                sem).wait()

    for d in range(_DD):
        t = t_ref[:, d:d + 1]
        logits = jax.lax.dot_general(
            s, wt_ref[d], (((1,), (0,)), ((), ())),
            preferred_element_type=jnp.float32)
        u = logits + bw[d:d + 1, :]
        du = u[:, 1:] - u[:, :-1]
        sdq = jax.lax.dot_general((du * du).astype(jnp.bfloat16), ones_nb1,
                                  dims, preferred_element_type=jnp.float32)
        sel = (blo < t) & (t <= bhi)
        selu = jax.lax.dot_general(
            jnp.where(sel, u, 0.0).astype(jnp.bfloat16), ones_nb, dims,
            preferred_element_type=jnp.float32)

        m = jnp.max(u, axis=1, keepdims=True)
        e = jnp.exp(u - m)
        z = jnp.sum(e, axis=1, keepdims=True)
        lz = m + jnp.log(z)
        logp = u - lz
        scr[d, :, :] = logp
        pltpu.make_async_copy(
            scr.at[d], logp_hbm.at[pl.ds(i * _BB, _BB), d, :], sem).start()
        salp = jax.lax.dot_general(jnp.abs(logp).astype(jnp.bfloat16),
                                   ones_nb, dims,
                                   preferred_element_type=jnp.float32)

        acc[0, :, d:d + 1] = selu[:, d:d + 1] - lz
        acc[1, :, d:d + 1] = sdq[:, d:d + 1]
        acc[2, :, d:d + 1] = salp[:, d:d + 1]

    glp = acc[0, :, :_DD]
    sdqp = acc[1, :, :_DD]
    salpp = acc[2, :, :_DD]
    spikp = (sdqp * (_NB / (_NB - 1.0))) / salpp
    mskp = (t_ref[...] != 0.0).astype(jnp.float32)
    lossp = (_LAMB * spikp - glp) * mskp
    nump = jnp.sum(lossp, axis=1, keepdims=True)
    denp = jnp.sum(mskp, axis=1, keepdims=True)
    per_ex = nump / jnp.maximum(denp, 1.0)
    part = jnp.sum(per_ex, axis=0, keepdims=True) * (1.0 / _B)
    loss_ref[0, :, :] = jnp.broadcast_to(part, (1, 128))

    @pl.when(i == nblk - 1)
    def _drain():
        for d in range(_DD):
            pltpu.make_async_copy(
                scr.at[d], logp_hbm.at[pl.ds(i * _BB, _BB), d, :],
                sem).wait()


@jax.jit
def _run(s, targets, wt, bt, bins2):
    nblk = _B // _BB
    logp, partials = pl.pallas_call(
        _fused_kernel,
        grid=(nblk,),
        in_specs=[
            pl.BlockSpec((_BB, _DM), lambda i: (i, 0)),
            pl.BlockSpec((_BB, _DD), lambda i: (i, 0)),
            pl.BlockSpec((_DD, _DM, _NB), lambda i: (0, 0, 0)),
            pl.BlockSpec((_DD, _NB), lambda i: (0, 0)),
            pl.BlockSpec((1, _NB + 1), lambda i: (0, 0)),
        ],
        out_specs=[
            pl.BlockSpec(memory_space=pltpu.MemorySpace.HBM),
            pl.BlockSpec((1, 1, 128), lambda i: (i, 0, 0)),
        ],
        out_shape=[
            jax.ShapeDtypeStruct((_B, _DD, _NB), jnp.float32),
            jax.ShapeDtypeStruct((nblk, 1, 128), jnp.float32),
        ],
        scratch_shapes=[
            pltpu.VMEM((_DD, _BB, _NB), jnp.float32),
            pltpu.VMEM((3, _BB, 128), jnp.float32),
            pltpu.SemaphoreType.DMA,
        ],
        compiler_params=pltpu.CompilerParams(
            dimension_semantics=("arbitrary",)),
    )(s, targets, wt, bt, bins2)
    loss = jnp.sum(partials[:, 0, 0])
    return loss, logp


def kernel(s, targets, W, b, bins):
    wt = jnp.transpose(W, (1, 2, 0))
    bt = jnp.transpose(b)
    bins2 = jnp.reshape(bins, (1, _NB + 1))
    return _run(s, targets, wt, bt, bins2)
